# baseline (speedup 1.0000x reference)
import functools

import jax
import jax.numpy as jnp
from jax import lax
from jax.experimental import pallas as pl
from jax.experimental.pallas import tpu as pltpu
from jax.experimental.pallas import tpu_sc as plsc

_NC = 2
_NS = 16
_NW = _NC * _NS
_SC_ROWS = 2048
_L = 16


def _tc_kernel(gids_ref, xt_ref, centers_ref, exc_ref, tgt_ref, out_ref,
               s1_acc, s2_acc, *, n_steps, blk, k_per_group, n_groups,
               inv_tau):
    i = pl.program_id(0)

    @pl.when(i == 0)
    def _init():
        s1_acc[:, :] = jnp.zeros_like(s1_acc)
        s2_acc[:, :] = jnp.zeros_like(s2_acc)

    xt = xt_ref[:, :]
    eb = jax.lax.dot_general(
        exc_ref[:, :], xt,
        dimension_numbers=(((1,), (0,)), ((), ())),
        preferred_element_type=jnp.float32)
    ee = jnp.exp(eb * inv_tau)

    row = i * blk + jax.lax.broadcasted_iota(jnp.int32, ee.shape, 0)
    row_cluster = row // k_per_group
    member = row_cluster == gids_ref[0]
    for g in range(1, n_groups):
        member = member | (row_cluster == gids_ref[g])

    s2_acc[:, :] += jnp.sum(ee, axis=0, keepdims=True)
    s1_acc[:, :] += jnp.sum(jnp.where(member, ee, 0.0), axis=0, keepdims=True)

    @pl.when(i == n_steps - 1)
    def _finalize():
        b = xt.shape[1]
        co = jax.lax.dot_general(
            centers_ref[:, :], xt,
            dimension_numbers=(((1,), (0,)), ((), ())),
            preferred_element_type=jnp.float32)
        se = jnp.sum(jnp.exp(co * inv_tau), axis=0)
        tgt = tgt_ref[0, :]
        rows = jax.lax.broadcasted_iota(jnp.int32, co.shape, 0)
        onehot = rows == tgt[None, :]
        out_t = jnp.sum(jnp.where(onehot, co, 0.0), axis=0)
        nce = -jnp.mean(out_t * inv_tau - jnp.log(se))
        out_ref[0, pl.ds(0, 32)] = s1_acc[0, :]
        out_ref[0, pl.ds(32, 32)] = s2_acc[0, :]
        out_ref[0, pl.ds(64, 32)] = jnp.full((32,), nce, jnp.float32)
        out_ref[0, pl.ds(96, 32)] = jnp.zeros((32,), jnp.float32)


def _tc_partials(inputs, targets, centers, excenters, tc_rows):
    b, d = inputs.shape
    c = centers.shape[0]
    _, k, _ = excenters.shape
    n_groups = b // k
    ck = excenters.shape[0] * k

    blk = 2048
    n_steps = tc_rows // blk

    exc2d = excenters.reshape(ck, d)
    xt = inputs.T
    gids = targets.reshape(n_groups, k)[:, 0]
    tgt2d = targets.reshape(1, b)

    grid_spec = pltpu.PrefetchScalarGridSpec(
        num_scalar_prefetch=1,
        grid=(n_steps,),
        in_specs=[
            pl.BlockSpec((d, b), lambda i, g: (0, 0)),
            pl.BlockSpec((c, d), lambda i, g: (0, 0)),
            pl.BlockSpec((blk, d), lambda i, g: (i, 0)),
            pl.BlockSpec((1, b), lambda i, g: (0, 0)),
        ],
        out_specs=pl.BlockSpec((1, 128), lambda i, g: (0, 0)),
        scratch_shapes=[
            pltpu.VMEM((1, b), jnp.float32),
            pltpu.VMEM((1, b), jnp.float32),
        ],
    )

    fn = functools.partial(
        _tc_kernel, n_steps=n_steps, blk=blk, k_per_group=k,
        n_groups=n_groups, inv_tau=20.0)

    return pl.pallas_call(
        fn,
        grid_spec=grid_spec,
        out_shape=jax.ShapeDtypeStruct((1, 128), jnp.float32),
    )(gids, xt, centers, exc2d, tgt2d)


def _sc_body(exc_ref, x_ref, gids_ref, out_ref, x_v, rows_v, gid_v,
             stage_v, *, d, b, k_per_group, n_groups, inv_tau, row0):
    wid = lax.axis_index("s") * _NC + lax.axis_index("c")
    worker_rows = _SC_ROWS // _NW
    my_row0 = row0 + wid * worker_rows

    pltpu.sync_copy(x_ref, x_v)
    pltpu.sync_copy(gids_ref, gid_v)

    _RB = 8
    _U = 4
    n_blocks = worker_rows // _RB
    n_chunks = d // _L
    lane_ids = lax.iota(jnp.int32, 16)

    def block_body(blk_i, carry):
        pltpu.sync_copy(exc_ref.at[pl.ds(my_row0 + blk_i * _RB, _RB), :],
                        rows_v)
        zero = jnp.zeros((_L,), jnp.float32)
        gv = gid_v[pl.ds(0, 16)]
        s1lo, s1hi, s2lo, s2hi = carry

        for pb in range(_RB // 2):
            dots = [zero, zero, zero, zero]
            for half in range(2):
                accs = [zero for _ in range(2 * _L)]

                def chunk_body(j, acc, pb=pb, half=half):
                    acc = list(acc)
                    for u in range(_U):
                        off = (j * _U + u) * _L
                        ve_a = rows_v[2 * pb, pl.ds(off, _L)]
                        ve_b = rows_v[2 * pb + 1, pl.ds(off, _L)]
                        for ii in range(_L):
                            xi = x_v[half * _L + ii, pl.ds(off, _L)]
                            acc[2 * ii] = acc[2 * ii] + ve_a * xi
                            acc[2 * ii + 1] = acc[2 * ii + 1] + ve_b * xi
                    return tuple(acc)

                accs = lax.fori_loop(0, n_chunks // _U, chunk_body,
                                     tuple(accs))
                for ii in range(_L):
                    sel = lane_ids == ii
                    da = jnp.sum(accs[2 * ii], axis=0)
                    db = jnp.sum(accs[2 * ii + 1], axis=0)
                    dots[half] = jnp.where(sel, da, dots[half])
                    dots[2 + half] = jnp.where(sel, db, dots[2 + half])

            for r in range(2):
                row_id = my_row0 + blk_i * _RB + 2 * pb + r
                cl = row_id // k_per_group
                mf = jnp.sum(jnp.where(gv == cl, 1.0, 0.0), axis=0)
                elo = jnp.exp(dots[2 * r + 0] * inv_tau)
                ehi = jnp.exp(dots[2 * r + 1] * inv_tau)
                s2lo = s2lo + elo
                s2hi = s2hi + ehi
                s1lo = s1lo + elo * mf
                s1hi = s1hi + ehi * mf
        return (s1lo, s1hi, s2lo, s2hi)

    zero = jnp.zeros((_L,), jnp.float32)
    s1lo, s1hi, s2lo, s2hi = lax.fori_loop(
        0, n_blocks, block_body, (zero, zero, zero, zero))

    stage_v[pl.ds(0, _L)] = s1lo
    stage_v[pl.ds(_L, _L)] = s1hi
    stage_v[pl.ds(2 * _L, _L)] = s2lo
    stage_v[pl.ds(3 * _L, _L)] = s2hi
    for q in range(4, 8):
        stage_v[pl.ds(q * _L, _L)] = zero
    pltpu.sync_copy(stage_v, out_ref.at[pl.ds(wid * 128, 128)])


def _sc_partials(exc2d, inputs, gids16, row0):
    ck, d = exc2d.shape
    b = inputs.shape[0]
    mesh = plsc.VectorSubcoreMesh(core_axis_name="c", subcore_axis_name="s")
    run = pl.kernel(
        functools.partial(_sc_body, d=d, b=b, k_per_group=16, n_groups=2,
                          inv_tau=20.0, row0=row0),
        out_type=jax.ShapeDtypeStruct((_NW * 128,), jnp.float32),
        mesh=mesh,
        scratch_types=[
            pltpu.VMEM((b, d), jnp.float32),
            pltpu.VMEM((8, d), jnp.float32),
            pltpu.VMEM((16,), jnp.int32),
            pltpu.VMEM((128,), jnp.float32),
        ],
        compiler_params=pltpu.CompilerParams(use_tc_tiling_on_sc=True,
                                             needs_layout_passes=False),
    )
    return run(exc2d, inputs, gids16)


def _combine_kernel(tc_ref, sc_ref, out_ref):
    tc = tc_ref[:, :]
    scs = jnp.sum(sc_ref[:, :], axis=0, keepdims=True)
    s1 = tc[:, 0:32] + scs[:, 0:32]
    s2 = tc[:, 32:64] + scs[:, 32:64]
    nce = jnp.sum(tc[:, 64:96]) * (1.0 / 32.0)
    l2 = jnp.mean(jnp.log(s2) - jnp.log(s1))
    out_ref[0, 0] = nce + l2


def _combine(tc_out, sc_out):
    out = pl.pallas_call(
        _combine_kernel,
        out_specs=pl.BlockSpec(memory_space=pltpu.SMEM),
        out_shape=jax.ShapeDtypeStruct((1, 1), jnp.float32),
    )(tc_out, sc_out)
    return out[0, 0]


def kernel(inputs, idxs, targets, cams, centers, excenters):
    del idxs, cams
    b, d = inputs.shape
    _, k, _ = excenters.shape
    n_groups = b // k
    ck = excenters.shape[0] * k
    tc_rows = ck - _SC_ROWS

    exc2d = excenters.reshape(ck, d)
    gids = targets.reshape(n_groups, k)[:, 0]
    gids16 = jnp.pad(gids, (0, 16 - n_groups), constant_values=-1)

    tc_out = _tc_partials(inputs, targets, centers, excenters, tc_rows)
    sc_out = _sc_partials(exc2d, inputs, gids16, tc_rows)
    return _combine(tc_out, sc_out.reshape(_NW, 128))


# speedup vs baseline: 2.7962x; 2.7962x over previous
import functools

import jax
import jax.numpy as jnp
from jax.experimental import pallas as pl
from jax.experimental.pallas import tpu as pltpu


def _loss_kernel(gids_ref, xt_ref, centers_ref, exc_ref, tgt_ref, out_ref,
                 s1_acc, s2_acc, *, n_steps, blk, k_per_group, n_groups,
                 inv_tau):
    i = pl.program_id(0)

    @pl.when(i == 0)
    def _init():
        s1_acc[:, :] = jnp.zeros_like(s1_acc)
        s2_acc[:, :] = jnp.zeros_like(s2_acc)

    xt = xt_ref[:, :]
    eb = jax.lax.dot_general(
        exc_ref[:, :], xt,
        dimension_numbers=(((1,), (0,)), ((), ())),
        preferred_element_type=jnp.float32)
    ee = jnp.exp(eb * inv_tau)

    row = i * blk + jax.lax.broadcasted_iota(jnp.int32, ee.shape, 0)
    row_cluster = row // k_per_group
    member = row_cluster == gids_ref[0]
    for g in range(1, n_groups):
        member = member | (row_cluster == gids_ref[g])

    s2_acc[:, :] += jnp.sum(ee, axis=0, keepdims=True)
    s1_acc[:, :] += jnp.sum(jnp.where(member, ee, 0.0), axis=0, keepdims=True)

    @pl.when(i == n_steps - 1)
    def _finalize():
        co = jax.lax.dot_general(
            centers_ref[:, :], xt,
            dimension_numbers=(((1,), (0,)), ((), ())),
            preferred_element_type=jnp.float32)
        se = jnp.sum(jnp.exp(co * inv_tau), axis=0)
        tgt = tgt_ref[0, :]
        rows = jax.lax.broadcasted_iota(jnp.int32, co.shape, 0)
        onehot = rows == tgt[None, :]
        out_t = jnp.sum(jnp.where(onehot, co, 0.0), axis=0)
        nce = -jnp.mean(out_t * inv_tau - jnp.log(se))
        l2 = jnp.mean(jnp.log(s2_acc[0, :]) - jnp.log(s1_acc[0, :]))
        out_ref[0, 0] = nce + l2


def kernel(inputs, idxs, targets, cams, centers, excenters):
    del idxs, cams
    b, d = inputs.shape
    c = centers.shape[0]
    _, k, _ = excenters.shape
    n_groups = b // k
    ck = excenters.shape[0] * k

    blk = 2048
    n_steps = ck // blk

    exc2d = excenters.reshape(ck, d)
    xt = inputs.T
    gids = targets.reshape(n_groups, k)[:, 0]
    tgt2d = targets.reshape(1, b)

    grid_spec = pltpu.PrefetchScalarGridSpec(
        num_scalar_prefetch=1,
        grid=(n_steps,),
        in_specs=[
            pl.BlockSpec((d, b), lambda i, g: (0, 0)),
            pl.BlockSpec((c, d), lambda i, g: (0, 0)),
            pl.BlockSpec((blk, d), lambda i, g: (i, 0)),
            pl.BlockSpec((1, b), lambda i, g: (0, 0)),
        ],
        out_specs=pl.BlockSpec(memory_space=pltpu.SMEM),
        scratch_shapes=[
            pltpu.VMEM((1, b), jnp.float32),
            pltpu.VMEM((1, b), jnp.float32),
        ],
    )

    fn = functools.partial(
        _loss_kernel, n_steps=n_steps, blk=blk, k_per_group=k,
        n_groups=n_groups, inv_tau=20.0)

    out = pl.pallas_call(
        fn,
        grid_spec=grid_spec,
        out_shape=jax.ShapeDtypeStruct((1, 1), jnp.float32),
    )(gids, xt, centers, exc2d, tgt2d)
    return out[0, 0]


# speedup vs baseline: 2.8890x; 1.0332x over previous
import functools

import jax
import jax.numpy as jnp
from jax.experimental import pallas as pl
from jax.experimental.pallas import tpu as pltpu


def _loss_kernel(tgt_s_ref, x_ref, centers_ref, exc_ref, tgt_ref, out_ref,
                 xt_s, s1_acc, s2_acc, nce_s, *, n_steps, blk, k_per_group,
                 n_groups, inv_tau):
    i = pl.program_id(0)

    @pl.when(i == 0)
    def _prologue():
        s1_acc[:, :] = jnp.zeros_like(s1_acc)
        s2_acc[:, :] = jnp.zeros_like(s2_acc)
        x = x_ref[:, :]
        xt_s[:, :] = x.T
        co = jax.lax.dot_general(
            centers_ref[:, :], x,
            dimension_numbers=(((1,), (1,)), ((), ())),
            preferred_element_type=jnp.float32)
        se = jnp.sum(jnp.exp(co * inv_tau), axis=0)
        tgt = tgt_ref[0, :]
        rows = jax.lax.broadcasted_iota(jnp.int32, co.shape, 0)
        onehot = rows == tgt[None, :]
        out_t = jnp.sum(jnp.where(onehot, co, 0.0), axis=0)
        nce_s[0, 0] = -jnp.mean(out_t * inv_tau - jnp.log(se))

    eb = jax.lax.dot_general(
        exc_ref[:, :], xt_s[:, :],
        dimension_numbers=(((1,), (0,)), ((), ())),
        preferred_element_type=jnp.float32)
    ee = jnp.exp(eb * inv_tau)

    row = i * blk + jax.lax.broadcasted_iota(jnp.int32, ee.shape, 0)
    row_cluster = row // k_per_group
    member = row_cluster == tgt_s_ref[0]
    for g in range(1, n_groups):
        member = member | (row_cluster == tgt_s_ref[g * k_per_group])

    s2_acc[:, :] += jnp.sum(ee, axis=0, keepdims=True)
    s1_acc[:, :] += jnp.sum(jnp.where(member, ee, 0.0), axis=0, keepdims=True)

    @pl.when(i == n_steps - 1)
    def _finalize():
        l2 = jnp.mean(jnp.log(s2_acc[0, :]) - jnp.log(s1_acc[0, :]))
        out_ref[0, 0] = nce_s[0, 0] + l2


def kernel(inputs, idxs, targets, cams, centers, excenters):
    del idxs, cams
    b, d = inputs.shape
    c = centers.shape[0]
    _, k, _ = excenters.shape
    n_groups = b // k
    ck = excenters.shape[0] * k

    blk = 1024
    n_steps = ck // blk

    exc2d = excenters.reshape(ck, d)
    tgt2d = targets.reshape(1, b)

    grid_spec = pltpu.PrefetchScalarGridSpec(
        num_scalar_prefetch=1,
        grid=(n_steps,),
        in_specs=[
            pl.BlockSpec((b, d), lambda i, g: (0, 0)),
            pl.BlockSpec((c, d), lambda i, g: (0, 0)),
            pl.BlockSpec((blk, d), lambda i, g: (i, 0)),
            pl.BlockSpec((1, b), lambda i, g: (0, 0)),
        ],
        out_specs=pl.BlockSpec(memory_space=pltpu.SMEM),
        scratch_shapes=[
            pltpu.VMEM((d, b), jnp.float32),
            pltpu.VMEM((1, b), jnp.float32),
            pltpu.VMEM((1, b), jnp.float32),
            pltpu.SMEM((1, 1), jnp.float32),
        ],
    )

    fn = functools.partial(
        _loss_kernel, n_steps=n_steps, blk=blk, k_per_group=k,
        n_groups=n_groups, inv_tau=20.0)

    out = pl.pallas_call(
        fn,
        grid_spec=grid_spec,
        out_shape=jax.ShapeDtypeStruct((1, 1), jnp.float32),
    )(targets, inputs, centers, exc2d, tgt2d)
    return out[0, 0]
